# baseline (device time: 29420 ns/iter reference)
import jax
import jax.numpy as jnp
from jax import lax
from jax.experimental import pallas as pl
from jax.experimental.pallas import tpu as pltpu

N_DEV = 16
N_STEPS = 4
N_IDX = 512
ROWS_PER = 2048
D = 256


def kernel(table, idx):
    idx2 = idx.reshape(N_IDX, 1)

    def body(table_ref, idx_ref, out_ref, recv_ref, send_sems, recv_sems):
        my = lax.axis_index("i")

        barrier_sem = pltpu.get_barrier_semaphore()
        for s in range(N_STEPS):
            pl.semaphore_signal(
                barrier_sem,
                inc=1,
                device_id=(my ^ (1 << s),),
                device_id_type=pl.DeviceIdType.MESH,
            )

        local = idx_ref[:, :] - my * ROWS_PER
        cols = lax.broadcasted_iota(jnp.int32, (N_IDX, ROWS_PER), 1)
        onehot = (cols == local).astype(jnp.bfloat16)
        partial = lax.dot_general(
            onehot,
            table_ref[:, :].astype(jnp.bfloat16),
            (((1,), (0,)), ((), ())),
            preferred_element_type=jnp.float32,
        )
        out_ref[:, :] = partial.astype(jnp.bfloat16)

        pl.semaphore_wait(barrier_sem, N_STEPS)

        for s in range(N_STEPS):
            partner = my ^ (1 << s)
            rdma = pltpu.make_async_remote_copy(
                src_ref=out_ref,
                dst_ref=recv_ref.at[s],
                send_sem=send_sems.at[s],
                recv_sem=recv_sems.at[s],
                device_id=(partner,),
                device_id_type=pl.DeviceIdType.MESH,
            )
            rdma.start()
            rdma.wait()
            out_ref[:, :] = out_ref[:, :] + recv_ref[s, :, :]

    return pl.pallas_call(
        body,
        out_shape=jax.ShapeDtypeStruct((N_IDX, D), jnp.bfloat16),
        in_specs=[
            pl.BlockSpec(memory_space=pltpu.VMEM),
            pl.BlockSpec(memory_space=pltpu.VMEM),
        ],
        out_specs=pl.BlockSpec(memory_space=pltpu.VMEM),
        scratch_shapes=[
            pltpu.VMEM((N_STEPS, N_IDX, D), jnp.bfloat16),
            pltpu.SemaphoreType.DMA((N_STEPS,)),
            pltpu.SemaphoreType.DMA((N_STEPS,)),
        ],
        compiler_params=pltpu.CompilerParams(collective_id=0),
    )(table, idx2)


# device time: 24617 ns/iter; 1.1951x vs baseline; 1.1951x over previous
import jax
import jax.numpy as jnp
from jax import lax
from jax.experimental import pallas as pl
from jax.experimental.pallas import tpu as pltpu

N_DEV = 16
N_STEPS = 4
N_IDX = 512
ROWS_PER = 2048
D = 256


def kernel(table, idx):
    idx2 = idx.reshape(N_IDX, 1)

    def body(table_ref, idx_ref, out_ref, recv_ref, send_sems, recv_sems):
        my = lax.axis_index("i")

        barrier_sem = pltpu.get_barrier_semaphore()
        for s in range(N_STEPS):
            pl.semaphore_signal(
                barrier_sem,
                inc=1,
                device_id=(my ^ (1 << s),),
                device_id_type=pl.DeviceIdType.MESH,
            )

        local = idx_ref[:, :] - my * ROWS_PER
        cols = lax.broadcasted_iota(jnp.int32, (N_IDX, ROWS_PER), 1)
        onehot = (cols == local).astype(jnp.bfloat16)
        partial = lax.dot_general(
            onehot,
            table_ref[:, :].astype(jnp.bfloat16),
            (((1,), (0,)), ((), ())),
            preferred_element_type=jnp.float32,
        )
        out_ref[:, :] = partial.astype(jnp.bfloat16)

        pl.semaphore_wait(barrier_sem, N_STEPS)

        half = N_IDX // 2
        for s in range(N_STEPS):
            rails = []
            for r, step_dim in enumerate((s, N_STEPS - 1 - s)):
                partner = my ^ (1 << step_dim)
                rdma = pltpu.make_async_remote_copy(
                    src_ref=out_ref.at[pl.ds(r * half, half), :],
                    dst_ref=recv_ref.at[s, r],
                    send_sem=send_sems.at[s, r],
                    recv_sem=recv_sems.at[s, r],
                    device_id=(partner,),
                    device_id_type=pl.DeviceIdType.MESH,
                )
                rdma.start()
                rails.append(rdma)
            for r, rdma in enumerate(rails):
                rdma.wait()
                rows = pl.ds(r * half, half)
                out_ref[rows, :] = out_ref[rows, :] + recv_ref[s, r, :, :]

    return pl.pallas_call(
        body,
        out_shape=jax.ShapeDtypeStruct((N_IDX, D), jnp.bfloat16),
        in_specs=[
            pl.BlockSpec(memory_space=pltpu.VMEM),
            pl.BlockSpec(memory_space=pltpu.VMEM),
        ],
        out_specs=pl.BlockSpec(memory_space=pltpu.VMEM),
        scratch_shapes=[
            pltpu.VMEM((N_STEPS, 2, N_IDX // 2, D), jnp.bfloat16),
            pltpu.SemaphoreType.DMA((N_STEPS, 2)),
            pltpu.SemaphoreType.DMA((N_STEPS, 2)),
        ],
        compiler_params=pltpu.CompilerParams(collective_id=0),
    )(table, idx2)


# device time: 22299 ns/iter; 1.3193x vs baseline; 1.1040x over previous
import jax
import jax.numpy as jnp
from jax import lax
from jax.experimental import pallas as pl
from jax.experimental.pallas import tpu as pltpu

N_DEV = 16
N_STEPS = 4
N_IDX = 512
ROWS_PER = 2048
D = 256
HALF = N_IDX // 2

MASKS_A = (1, 3, 4, 8)
MASKS_B = (8, 4, 3, 1)


def kernel(table, idx):
    idx2 = idx.reshape(N_IDX, 1)

    def body(table_ref, idx_ref, out_ref, recv_ref, send_sems, recv_sems):
        my = lax.axis_index("i")

        barrier_sem = pltpu.get_barrier_semaphore()
        for m in MASKS_A:
            pl.semaphore_signal(
                barrier_sem,
                inc=1,
                device_id=(my ^ m,),
                device_id_type=pl.DeviceIdType.MESH,
            )

        table_bf16 = table_ref[:, :].astype(jnp.bfloat16)

        def partial_half(r):
            local = idx_ref[pl.ds(r * HALF, HALF), :] - my * ROWS_PER
            cols = lax.broadcasted_iota(jnp.int32, (HALF, ROWS_PER), 1)
            onehot = (cols == local).astype(jnp.bfloat16)
            acc = lax.dot_general(
                onehot,
                table_bf16,
                (((1,), (0,)), ((), ())),
                preferred_element_type=jnp.float32,
            )
            out_ref[pl.ds(r * HALF, HALF), :] = acc.astype(jnp.bfloat16)

        def start(rail, s, masks):
            rdma = pltpu.make_async_remote_copy(
                src_ref=out_ref.at[pl.ds(rail * HALF, HALF), :],
                dst_ref=recv_ref.at[s, rail],
                send_sem=send_sems.at[s, rail],
                recv_sem=recv_sems.at[s, rail],
                device_id=(my ^ masks[s],),
                device_id_type=pl.DeviceIdType.MESH,
            )
            rdma.start()
            return rdma

        partial_half(0)
        pl.semaphore_wait(barrier_sem, N_STEPS)
        rdma_a = start(0, 0, MASKS_A)
        partial_half(1)
        rdma_b = start(1, 0, MASKS_B)

        for s in range(N_STEPS):
            rdma_a.wait()
            rows_a = pl.ds(0, HALF)
            out_ref[rows_a, :] = out_ref[rows_a, :] + recv_ref[s, 0, :, :]
            if s + 1 < N_STEPS:
                rdma_a = start(0, s + 1, MASKS_A)
            rdma_b.wait()
            rows_b = pl.ds(HALF, HALF)
            out_ref[rows_b, :] = out_ref[rows_b, :] + recv_ref[s, 1, :, :]
            if s + 1 < N_STEPS:
                rdma_b = start(1, s + 1, MASKS_B)

    return pl.pallas_call(
        body,
        out_shape=jax.ShapeDtypeStruct((N_IDX, D), jnp.bfloat16),
        in_specs=[
            pl.BlockSpec(memory_space=pltpu.VMEM),
            pl.BlockSpec(memory_space=pltpu.VMEM),
        ],
        out_specs=pl.BlockSpec(memory_space=pltpu.VMEM),
        scratch_shapes=[
            pltpu.VMEM((N_STEPS, 2, HALF, D), jnp.bfloat16),
            pltpu.SemaphoreType.DMA((N_STEPS, 2)),
            pltpu.SemaphoreType.DMA((N_STEPS, 2)),
        ],
        compiler_params=pltpu.CompilerParams(collective_id=0),
    )(table, idx2)
